# baseline (speedup 1.0000x reference)
import jax
import jax.numpy as jnp
from jax.experimental import pallas as pl
from jax.experimental.pallas import tpu as pltpu

_T = 2048
_D = 1024
_E = 8
_CH = 256
_NC = _T // _CH


def _fused_kernel(x_ref, wa_ref, bias_ref, ws_ref, bs_ref, out_ref,
                  xb_ref, w_ref, wsb_ref):
    e = pl.program_id(0)

    @pl.when(e == 0)
    def _prologue():
        for m in range(_NC):
            sl = pl.ds(m * _CH, _CH)
            x32 = x_ref[sl, :]
            logits = jnp.dot(x32, wa_ref[...],
                             preferred_element_type=jnp.float32) + bias_ref[...]
            w_ref[sl, :] = jax.nn.softmax(logits, axis=-1)
            xb_ref[sl, :] = x32.astype(jnp.bfloat16)

    wsb_ref[...] = ws_ref[0].astype(jnp.bfloat16)
    b = bs_ref[0]
    for m in range(_NC):
        sl = pl.ds(m * _CH, _CH)
        h = jnp.dot(xb_ref[sl, :], wsb_ref[...],
                    preferred_element_type=jnp.float32)
        h = jnp.tanh(h + b)
        w = w_ref[sl, :]
        lane = jax.lax.broadcasted_iota(jnp.int32, w.shape, 1)
        we = jnp.sum(jnp.where(lane == e, w, 0.0), axis=1, keepdims=True)
        contrib = we * h

        @pl.when(e == 0)
        def _init():
            out_ref[sl, :] = contrib

        @pl.when(e != 0)
        def _acc():
            out_ref[sl, :] = out_ref[sl, :] + contrib


def kernel(x, W_attn, b_attn, adaptive_bias, W_s, b_s):
    bias = (b_attn + adaptive_bias).reshape(1, _E)
    return pl.pallas_call(
        _fused_kernel,
        grid=(_E,),
        in_specs=[
            pl.BlockSpec((_T, _D), lambda e: (0, 0)),
            pl.BlockSpec((_D, _E), lambda e: (0, 0)),
            pl.BlockSpec((1, _E), lambda e: (0, 0)),
            pl.BlockSpec((1, _D, _D), lambda e: (e, 0, 0)),
            pl.BlockSpec((1, 1, _D), lambda e: (e, 0, 0)),
        ],
        out_specs=pl.BlockSpec((_T, _D), lambda e: (0, 0)),
        out_shape=jax.ShapeDtypeStruct((_T, _D), jnp.float32),
        scratch_shapes=[
            pltpu.VMEM((_T, _D), jnp.bfloat16),
            pltpu.VMEM((_T, _E), jnp.float32),
            pltpu.VMEM((_D, _D), jnp.bfloat16),
        ],
        compiler_params=pltpu.CompilerParams(
            dimension_semantics=("arbitrary",),
        ),
    )(x, W_attn, bias, W_s, b_s.reshape(_E, 1, _D))


# speedup vs baseline: 1.1521x; 1.1521x over previous
import jax
import jax.numpy as jnp
from jax.experimental import pallas as pl
from jax.experimental.pallas import tpu as pltpu

_T = 2048
_D = 1024
_E = 8
_NH = 2
_HW = _D // _NH


def _fused_kernel(x_ref, wa_ref, bias_ref, ws_ref, bs_ref, out_ref,
                  xb_ref, w_ref):
    s = pl.program_id(0)
    e = s // _NH

    @pl.when(s == 0)
    def _prologue():
        x32 = x_ref[...]
        logits = jnp.dot(x32, wa_ref[...],
                         preferred_element_type=jnp.float32) + bias_ref[...]
        w_ref[...] = jax.nn.softmax(logits, axis=-1)
        xb_ref[...] = x32.astype(jnp.bfloat16)

    wsb = ws_ref[0].astype(jnp.bfloat16)
    h = jnp.dot(xb_ref[...], wsb, preferred_element_type=jnp.float32)
    h = jnp.tanh(h + bs_ref[0])
    w = w_ref[...]
    lane = jax.lax.broadcasted_iota(jnp.int32, w.shape, 1)
    we = jnp.sum(jnp.where(lane == e, w, 0.0), axis=1, keepdims=True)
    contrib = we * h
    half = s % _NH
    csl = pl.ds(half * _HW, _HW)

    @pl.when(e == 0)
    def _init():
        out_ref[:, csl] = contrib

    @pl.when(e != 0)
    def _acc():
        out_ref[:, csl] = out_ref[:, csl] + contrib


def kernel(x, W_attn, b_attn, adaptive_bias, W_s, b_s):
    bias = (b_attn + adaptive_bias).reshape(1, _E)
    return pl.pallas_call(
        _fused_kernel,
        grid=(_E * _NH,),
        in_specs=[
            pl.BlockSpec((_T, _D), lambda s: (0, 0)),
            pl.BlockSpec((_D, _E), lambda s: (0, 0)),
            pl.BlockSpec((1, _E), lambda s: (0, 0)),
            pl.BlockSpec((1, _D, _HW),
                         lambda s: (s // _NH, 0, s % _NH)),
            pl.BlockSpec((1, 1, _HW),
                         lambda s: (s // _NH, 0, s % _NH)),
        ],
        out_specs=pl.BlockSpec((_T, _D), lambda s: (0, 0)),
        out_shape=jax.ShapeDtypeStruct((_T, _D), jnp.float32),
        scratch_shapes=[
            pltpu.VMEM((_T, _D), jnp.bfloat16),
            pltpu.VMEM((_T, _E), jnp.float32),
        ],
        compiler_params=pltpu.CompilerParams(
            dimension_semantics=("arbitrary",),
        ),
    )(x, W_attn, bias, W_s, b_s.reshape(_E, 1, _D))
